# Initial kernel scaffold; baseline (speedup 1.0000x reference)
import jax
import jax.numpy as jnp
from jax.experimental import pallas as pl


def kernel(u, i, uemb, iemb, ubias, ibias, gbias):
    raise NotImplementedError("write your pallas kernel here")


# speedup vs baseline: 1.0645x; 1.0645x over previous
import functools

import jax
import jax.numpy as jnp
from jax import lax
from jax.experimental import pallas as pl
from jax.experimental.pallas import tpu as pltpu
from jax.experimental.pallas import tpu_sc as plsc

B = 16384
D = 128
NC = 2
NS = 16
NW = NC * NS
BPW = B // NW
CHUNK = 128
NCHUNK = BPW // CHUNK
LANES = 16


def _svd_body(u_hbm, i_hbm, uemb_hbm, iemb_hbm, ubias_hbm, ibias_hbm,
              gbias_hbm, out_hbm,
              uix, iix, ub, ib, gb, ubuf0, ubuf1, ibuf0, ibuf1, out_v,
              sem0, sem1, bsem):
    c = lax.axis_index("c")
    s = lax.axis_index("s")
    wid = s * NC + c
    base = wid * BPW

    pltpu.sync_copy(u_hbm.at[pl.ds(base, BPW)], uix)
    pltpu.sync_copy(i_hbm.at[pl.ds(base, BPW)], iix)
    pltpu.sync_copy(gbias_hbm, gb.at[pl.ds(0, 1)])

    bias_handles = []
    for ci in range(NCHUNK):
        sl = pl.ds(ci * CHUNK, CHUNK)
        bias_handles.append(
            pltpu.async_copy(ubias_hbm.at[uix.at[sl]], ub.at[sl], bsem))
        bias_handles.append(
            pltpu.async_copy(ibias_hbm.at[iix.at[sl]], ib.at[sl], bsem))

    ubufs = (ubuf0, ubuf1)
    ibufs = (ibuf0, ibuf1)
    sems = (sem0, sem1)

    def start(ci):
        slot = ci % 2
        sl = pl.ds(ci * CHUNK, CHUNK)
        hu = pltpu.async_copy(uemb_hbm.at[uix.at[sl]], ubufs[slot], sems[slot])
        hi = pltpu.async_copy(iemb_hbm.at[iix.at[sl]], ibufs[slot], sems[slot])
        return hu, hi

    handles = [None, None]
    handles[0] = start(0)
    for ci in range(NCHUNK):
        slot = ci % 2
        if ci + 1 < NCHUNK:
            handles[(ci + 1) % 2] = start(ci + 1)
        hu, hi = handles[slot]
        hu.wait()
        hi.wait()
        ubuf = ubufs[slot]
        ibuf = ibufs[slot]
        out_base = ci * CHUNK
        lane_iota = lax.iota(jnp.int32, LANES)

        def group_body(g, _):
            rowbase = g * LANES

            def row_body(r, vec):
                row = rowbase + r
                acc = ubuf[row, pl.ds(0, LANES)] * ibuf[row, pl.ds(0, LANES)]
                for j in range(1, D // LANES):
                    sl2 = pl.ds(j * LANES, LANES)
                    acc = acc + ubuf[row, sl2] * ibuf[row, sl2]
                return jnp.where(lane_iota == r, jnp.sum(acc), vec)

            vec = lax.fori_loop(0, LANES, row_body,
                                jnp.zeros((LANES,), jnp.float32), unroll=2)
            out_v[pl.ds(out_base + rowbase, LANES)] = vec
            return 0

        lax.fori_loop(0, CHUNK // LANES, group_body, 0)

    for h in bias_handles:
        h.wait()
    gbs = gb[pl.ds(0, LANES)][0]
    for g in range(BPW // LANES):
        sl = pl.ds(g * LANES, LANES)
        out_v[sl] = out_v[sl] + ub[sl] + ib[sl] + gbs

    pltpu.sync_copy(out_v, out_hbm.at[pl.ds(base, BPW)])


@functools.partial(jax.jit, donate_argnums=())
def kernel(u, i, uemb, iemb, ubias, ibias, gbias):
    mesh = plsc.VectorSubcoreMesh(core_axis_name="c", subcore_axis_name="s")
    run = pl.kernel(
        _svd_body,
        mesh=mesh,
        compiler_params=pltpu.CompilerParams(needs_layout_passes=False),
        out_type=jax.ShapeDtypeStruct((B,), jnp.float32),
        scratch_types=[
            pltpu.VMEM((BPW,), jnp.int32),
            pltpu.VMEM((BPW,), jnp.int32),
            pltpu.VMEM((BPW,), jnp.float32),
            pltpu.VMEM((BPW,), jnp.float32),
            pltpu.VMEM((LANES,), jnp.float32),
            pltpu.VMEM((CHUNK, D), jnp.float32),
            pltpu.VMEM((CHUNK, D), jnp.float32),
            pltpu.VMEM((CHUNK, D), jnp.float32),
            pltpu.VMEM((CHUNK, D), jnp.float32),
            pltpu.VMEM((BPW,), jnp.float32),
            pltpu.SemaphoreType.DMA,
            pltpu.SemaphoreType.DMA,
            pltpu.SemaphoreType.DMA,
        ],
    )
    return run(u, i, uemb, iemb, ubias.reshape(-1), ibias.reshape(-1), gbias)
